# baseline (speedup 1.0000x reference)
import functools

import jax
import jax.numpy as jnp
from jax import lax
from jax.experimental import pallas as pl
from jax.experimental.pallas import tpu as pltpu
from jax.experimental.pallas import tpu_sc as plsc

C = 128
W2 = C // 2
RES = 256
NC = 2
NS = 16
NW = NC * NS
CB = 64
NCHUNK = 50
PW = CB * NCHUNK
N_PAD = NW * PW


def _tri_body(t0, t1, t2, m0h, m1h, m2h, outh,
              m0v, m1v, m2v, idxv, wv, taps, outv0, outv1,
              sem0, sem1, osem0, osem1):
    wid = lax.axis_index("c") * NS + lax.axis_index("s")
    base = wid * PW

    pltpu.sync_copy(m0h.at[pl.ds(base, PW)], m0v)
    pltpu.sync_copy(m1h.at[pl.ds(base, PW)], m1v)
    pltpu.sync_copy(m2h.at[pl.ds(base, PW)], m2v)

    tables = (t0, t1, t2)
    sems = (sem0, sem1)
    outvs = (outv0, outv1)
    osems = (osem0, osem1)

    def compute_idx(ci, half):
        for g in range(CB // 16):
            s = ci * CB + g * 16
            mm0 = jnp.clip(m0v[pl.ds(s, 16)], 0.0, 1.0 - 1e-6) * RES
            mm1 = jnp.clip(m1v[pl.ds(s, 16)], 0.0, 1.0 - 1e-6) * RES
            mm2 = jnp.clip(m2v[pl.ds(s, 16)], 0.0, 1.0 - 1e-6) * RES
            proj = ((mm0, mm1), (mm1, mm2), (mm2, mm0))
            for k in range(3):
                px, py = proj[k]
                xi = px.astype(jnp.int32)
                yi = py.astype(jnp.int32)
                wx = px - xi.astype(jnp.float32)
                wy = py - yi.astype(jnp.float32)
                xok = xi < RES - 1
                yok = yi < RES - 1
                xs = jnp.where(xok, 1, 0)
                wx1 = jnp.where(xok, wx, 0.0)
                wy1 = jnp.where(yok, wy, 0.0)
                i00 = yi * RES + xi
                row = half * 3 + k
                idxv[row, pl.ds(0 * CB + g * 16, 16)] = i00
                idxv[row, pl.ds(1 * CB + g * 16, 16)] = i00 + xs
                wrow = (half * 12) + 4 * k
                gs = pl.ds(g * 16, 16)
                wv[wrow + 0, gs] = (1.0 - wx) * (1.0 - wy)
                wv[wrow + 1, gs] = wx1 * (1.0 - wy)
                wv[wrow + 2, gs] = (1.0 - wx) * wy1
                wv[wrow + 3, gs] = wx1 * wy1

    def fire(half):
        for k in range(3):
            for t in range(2):
                pltpu.async_copy(
                    tables[k].at[idxv.at[half * 3 + k, pl.ds(t * CB, CB)]],
                    taps.at[pl.ds(((half * 3 + k) * 2 + t) * CB, CB)],
                    sems[half])

    def drain(half):
        for k in range(3):
            for t in range(2):
                pltpu.make_async_copy(
                    tables[k].at[idxv.at[half * 3 + k, pl.ds(t * CB, CB)]],
                    taps.at[pl.ds(((half * 3 + k) * 2 + t) * CB, CB)],
                    sems[half]).wait()

    def combine(ci, half, par):
        ov = outvs[par]
        for g in range(CB // 16):
            rvec = lax.iota(jnp.int32, 16) + g * 16
            rows = [rvec + ((half * 3 + k) * 2 + t) * CB
                    for k in range(3) for t in range(2)]
            ws = [wv[half * 12 + j, pl.ds(g * 16, 16)] for j in range(12)]

            @plsc.parallel_loop(0, W2, unroll=1)
            def cbody(w, rows=rows, ws=ws):
                wvec = (jnp.full((16,), w, jnp.int32)
                        + lax.iota(jnp.int32, 16)) & (W2 - 1)
                fe = fo = None
                for k in range(3):
                    w00, w01, w10, w11 = (ws[4 * k + 0], ws[4 * k + 1],
                                          ws[4 * k + 2], ws[4 * k + 3])
                    rA, rB = rows[2 * k], rows[2 * k + 1]
                    v00 = plsc.load_gather(taps, [rA, wvec])
                    v10 = plsc.load_gather(taps, [rA, wvec + W2])
                    v01 = plsc.load_gather(taps, [rB, wvec])
                    v11 = plsc.load_gather(taps, [rB, wvec + W2])
                    e00, o00 = plsc.unpack(
                        plsc.bitcast(v00, jnp.bfloat16),
                        format=plsc.PackFormat.INTERLEAVED)
                    e10, o10 = plsc.unpack(
                        plsc.bitcast(v10, jnp.bfloat16),
                        format=plsc.PackFormat.INTERLEAVED)
                    e01, o01 = plsc.unpack(
                        plsc.bitcast(v01, jnp.bfloat16),
                        format=plsc.PackFormat.INTERLEAVED)
                    e11, o11 = plsc.unpack(
                        plsc.bitcast(v11, jnp.bfloat16),
                        format=plsc.PackFormat.INTERLEAVED)
                    fke = (w00 * e00 + w01 * e01) + (w10 * e10 + w11 * e11)
                    fko = (w00 * o00 + w01 * o01) + (w10 * o10 + w11 * o11)
                    fe = fke if fe is None else fe * fke
                    fo = fko if fo is None else fo * fko
                cvec0 = wvec << 1
                plsc.store_scatter(ov, [rvec, cvec0], fe)
                plsc.store_scatter(ov, [rvec, cvec0 + 1], fo)

        pltpu.async_copy(ov, outh.at[pl.ds(base + ci * CB, CB)],
                         osems[par])

    def out_wait(par):
        pltpu.make_async_copy(outvs[par], outh.at[pl.ds(base, CB)],
                              osems[par]).wait()

    compute_idx(0, 0)
    fire(0)

    def step_body(s, carry):
        c0 = 2 * s
        compute_idx(c0 + 1, 1)
        fire(1)
        drain(0)

        @pl.when(s > 0)
        def _():
            out_wait(0)

        combine(c0, 0, 0)

        @pl.when(s < NCHUNK // 2 - 1)
        def _():
            compute_idx(c0 + 2, 0)
            fire(0)

        drain(1)

        @pl.when(s > 0)
        def _():
            out_wait(1)

        combine(c0 + 1, 1, 1)
        return carry

    lax.fori_loop(0, NCHUNK // 2, step_body, 0)
    out_wait(0)
    out_wait(1)


_tri = pl.kernel(
    _tri_body,
    out_type=jax.ShapeDtypeStruct((N_PAD, C), jnp.float32),
    mesh=plsc.VectorSubcoreMesh(core_axis_name="c", subcore_axis_name="s"),
    compiler_params=pltpu.CompilerParams(needs_layout_passes=False,
                                         disable_bounds_checks=True),
    scratch_types=[
        pltpu.VMEM((PW,), jnp.float32),
        pltpu.VMEM((PW,), jnp.float32),
        pltpu.VMEM((PW,), jnp.float32),
        pltpu.VMEM((6, 2 * CB), jnp.int32),
        pltpu.VMEM((24, CB), jnp.float32),
        pltpu.VMEM((12 * CB, C), jnp.int32),
        pltpu.VMEM((CB, C), jnp.float32),
        pltpu.VMEM((CB, C), jnp.float32),
        pltpu.SemaphoreType.DMA,
        pltpu.SemaphoreType.DMA,
        pltpu.SemaphoreType.DMA,
        pltpu.SemaphoreType.DMA,
    ],
)


def kernel(mu, P0, P1, P2):
    n = mu.shape[0]

    def pack_table(p):
        t = jnp.transpose(p.reshape(C, RES * RES)).astype(jnp.bfloat16)
        ts = jnp.concatenate([t[RES:], t[-RES:]], axis=0)
        tv = jnp.concatenate([t, ts], axis=1)
        return lax.bitcast_convert_type(
            tv.reshape(RES * RES, C, 2), jnp.int32)

    tb0 = pack_table(P0)
    tb1 = pack_table(P1)
    tb2 = pack_table(P2)
    mt = jnp.pad(mu, ((0, N_PAD - n), (0, 0))).T
    out = _tri(tb0, tb1, tb2, mt[0], mt[1], mt[2])
    return out[:n]


# speedup vs baseline: 1.0003x; 1.0003x over previous
import functools

import jax
import jax.numpy as jnp
from jax import lax
from jax.experimental import pallas as pl
from jax.experimental.pallas import tpu as pltpu
from jax.experimental.pallas import tpu_sc as plsc

C = 128
W2 = C // 2
RES = 256
NC = 2
NS = 16
NW = NC * NS
CB = 64
NCHUNK = 50
PW = CB * NCHUNK
N_PAD = NW * PW


def _tri_body(t0, t1, t2, m0h, m1h, m2h, outh,
              m0v, m1v, m2v, idxv, wv, taps, outv0, outv1,
              sem0, sem1, osem0, osem1):
    wid = lax.axis_index("c") * NS + lax.axis_index("s")
    base = wid * PW

    pltpu.sync_copy(m0h.at[pl.ds(base, PW)], m0v)
    pltpu.sync_copy(m1h.at[pl.ds(base, PW)], m1v)
    pltpu.sync_copy(m2h.at[pl.ds(base, PW)], m2v)

    tables = (t0, t1, t2)
    sems = (sem0, sem1)
    outvs = (outv0, outv1)
    osems = (osem0, osem1)

    def compute_idx(ci, half):
        for g in range(CB // 16):
            s = ci * CB + g * 16
            mm0 = jnp.clip(m0v[pl.ds(s, 16)], 0.0, 1.0 - 1e-6) * RES
            mm1 = jnp.clip(m1v[pl.ds(s, 16)], 0.0, 1.0 - 1e-6) * RES
            mm2 = jnp.clip(m2v[pl.ds(s, 16)], 0.0, 1.0 - 1e-6) * RES
            proj = ((mm0, mm1), (mm1, mm2), (mm2, mm0))
            for k in range(3):
                px, py = proj[k]
                xi = px.astype(jnp.int32)
                yi = py.astype(jnp.int32)
                wx = px - xi.astype(jnp.float32)
                wy = py - yi.astype(jnp.float32)
                xok = xi < RES - 1
                yok = yi < RES - 1
                xs = jnp.where(xok, 1, 0)
                wx1 = jnp.where(xok, wx, 0.0)
                wy1 = jnp.where(yok, wy, 0.0)
                i00 = yi * RES + xi
                row = half * 3 + k
                idxv[row, pl.ds(0 * CB + g * 16, 16)] = i00
                idxv[row, pl.ds(1 * CB + g * 16, 16)] = i00 + xs
                wrow = (half * 12) + 4 * k
                gs = pl.ds(g * 16, 16)
                wv[wrow + 0, gs] = (1.0 - wx) * (1.0 - wy)
                wv[wrow + 1, gs] = wx1 * (1.0 - wy)
                wv[wrow + 2, gs] = (1.0 - wx) * wy1
                wv[wrow + 3, gs] = wx1 * wy1

    def fire(half):
        for k in range(3):
            for t in range(2):
                pltpu.async_copy(
                    tables[k].at[idxv.at[half * 3 + k, pl.ds(t * CB, CB)]],
                    taps.at[pl.ds(((half * 3 + k) * 2 + t) * CB, CB)],
                    sems[half])

    def drain(half):
        for k in range(3):
            for t in range(2):
                pltpu.make_async_copy(
                    tables[k].at[idxv.at[half * 3 + k, pl.ds(t * CB, CB)]],
                    taps.at[pl.ds(((half * 3 + k) * 2 + t) * CB, CB)],
                    sems[half]).wait()

    def combine(ci, half, par):
        ov = outvs[par]
        for g in range(CB // 16):
            rvec = lax.iota(jnp.int32, 16) + g * 16
            rows = [rvec + ((half * 3 + k) * 2 + t) * CB
                    for k in range(3) for t in range(2)]
            ws = [wv[half * 12 + j, pl.ds(g * 16, 16)] for j in range(12)]

            @plsc.parallel_loop(0, W2, unroll=1)
            def cbody(w, rows=rows, ws=ws):
                wvec = (jnp.full((16,), w, jnp.int32)
                        + lax.iota(jnp.int32, 16)) & (W2 - 1)
                fe = fo = None
                for k in range(3):
                    w00, w01, w10, w11 = (ws[4 * k + 0], ws[4 * k + 1],
                                          ws[4 * k + 2], ws[4 * k + 3])
                    rA, rB = rows[2 * k], rows[2 * k + 1]
                    v00 = plsc.load_gather(taps, [rA, wvec])
                    v10 = plsc.load_gather(taps, [rA, wvec + W2])
                    v01 = plsc.load_gather(taps, [rB, wvec])
                    v11 = plsc.load_gather(taps, [rB, wvec + W2])
                    e00, o00 = plsc.unpack(
                        plsc.bitcast(v00, jnp.bfloat16),
                        format=plsc.PackFormat.INTERLEAVED)
                    e10, o10 = plsc.unpack(
                        plsc.bitcast(v10, jnp.bfloat16),
                        format=plsc.PackFormat.INTERLEAVED)
                    e01, o01 = plsc.unpack(
                        plsc.bitcast(v01, jnp.bfloat16),
                        format=plsc.PackFormat.INTERLEAVED)
                    e11, o11 = plsc.unpack(
                        plsc.bitcast(v11, jnp.bfloat16),
                        format=plsc.PackFormat.INTERLEAVED)
                    fke = (w00 * e00 + w01 * e01) + (w10 * e10 + w11 * e11)
                    fko = (w00 * o00 + w01 * o01) + (w10 * o10 + w11 * o11)
                    fe = fke if fe is None else fe * fke
                    fo = fko if fo is None else fo * fko
                cvec0 = wvec << 1
                plsc.store_scatter(ov, [rvec, cvec0], fe)
                plsc.store_scatter(ov, [rvec, cvec0 + 1], fo)

        pltpu.async_copy(ov, outh.at[pl.ds(base + ci * CB, CB)],
                         osems[par])

    def out_wait(par):
        pltpu.make_async_copy(outvs[par], outh.at[pl.ds(base, CB)],
                              osems[par]).wait()

    compute_idx(0, 0)
    fire(0)

    def step_body(s, carry):
        c0 = 2 * s
        compute_idx(c0 + 1, 1)
        fire(1)
        drain(0)

        @pl.when(s > 0)
        def _():
            out_wait(0)

        combine(c0, 0, 0)

        @pl.when(s < NCHUNK // 2 - 1)
        def _():
            compute_idx(c0 + 2, 0)
            fire(0)

        drain(1)

        @pl.when(s > 0)
        def _():
            out_wait(1)

        combine(c0 + 1, 1, 1)
        return carry

    lax.fori_loop(0, NCHUNK // 2, step_body, 0)
    out_wait(0)
    out_wait(1)


_tri = pl.kernel(
    _tri_body,
    out_type=jax.ShapeDtypeStruct((N_PAD, C), jnp.float32),
    mesh=plsc.VectorSubcoreMesh(core_axis_name="c", subcore_axis_name="s"),
    compiler_params=pltpu.CompilerParams(needs_layout_passes=False,
                                         disable_bounds_checks=True),
    scratch_types=[
        pltpu.VMEM((PW,), jnp.float32),
        pltpu.VMEM((PW,), jnp.float32),
        pltpu.VMEM((PW,), jnp.float32),
        pltpu.VMEM((6, 2 * CB), jnp.int32),
        pltpu.VMEM((24, CB), jnp.float32),
        pltpu.VMEM((12 * CB, C), jnp.int32),
        pltpu.VMEM((CB, C), jnp.float32),
        pltpu.VMEM((CB, C), jnp.float32),
        pltpu.SemaphoreType.DMA,
        pltpu.SemaphoreType.DMA,
        pltpu.SemaphoreType.DMA,
        pltpu.SemaphoreType.DMA,
    ],
)


def kernel(mu, P0, P1, P2):
    n = mu.shape[0]

    def pack_table(p):
        t = jnp.transpose(p.reshape(C, RES * RES)).astype(jnp.bfloat16)
        tv = jnp.concatenate([t, jnp.roll(t, -RES, axis=0)], axis=1)
        return lax.bitcast_convert_type(
            tv.reshape(RES * RES, C, 2), jnp.int32)

    tb0 = pack_table(P0)
    tb1 = pack_table(P1)
    tb2 = pack_table(P2)
    mt = jnp.pad(mu, ((0, N_PAD - n), (0, 0))).T
    out = _tri(tb0, tb1, tb2, mt[0], mt[1], mt[2])
    return out[:n]


# speedup vs baseline: 2.6041x; 2.6032x over previous
import functools

import jax
import jax.numpy as jnp
from jax import lax
from jax.experimental import pallas as pl
from jax.experimental.pallas import tpu as pltpu
from jax.experimental.pallas import tpu_sc as plsc

C = 128
RES = 256
NC = 2
NS = 16
NW = NC * NS
CB = 32
NCHUNK = 100
PW = CB * NCHUNK
N_PAD = NW * PW


def _tri_body(t0, t1, t2, m0h, m1h, m2h, outh,
              m0v, m1v, m2v, idxv, wv, taps, outv0, outv1,
              sem0, sem1, osem0, osem1):
    wid = lax.axis_index("c") * NS + lax.axis_index("s")
    base = wid * PW

    pltpu.sync_copy(m0h.at[pl.ds(base, PW)], m0v)
    pltpu.sync_copy(m1h.at[pl.ds(base, PW)], m1v)
    pltpu.sync_copy(m2h.at[pl.ds(base, PW)], m2v)

    tables = (t0, t1, t2)
    sems = (sem0, sem1)
    outvs = (outv0, outv1)
    osems = (osem0, osem1)

    def compute_idx(ci, half):
        for g in range(CB // 16):
            s = ci * CB + g * 16
            mm0 = jnp.clip(m0v[pl.ds(s, 16)], 0.0, 1.0 - 1e-6) * RES
            mm1 = jnp.clip(m1v[pl.ds(s, 16)], 0.0, 1.0 - 1e-6) * RES
            mm2 = jnp.clip(m2v[pl.ds(s, 16)], 0.0, 1.0 - 1e-6) * RES
            proj = ((mm0, mm1), (mm1, mm2), (mm2, mm0))
            for k in range(3):
                px, py = proj[k]
                xi = px.astype(jnp.int32)
                yi = py.astype(jnp.int32)
                wx = px - xi.astype(jnp.float32)
                wy = py - yi.astype(jnp.float32)
                xok = xi < RES - 1
                yok = yi < RES - 1
                xs = jnp.where(xok, 1, 0)
                ys = jnp.where(yok, RES, 0)
                wx1 = jnp.where(xok, wx, 0.0)
                wy1 = jnp.where(yok, wy, 0.0)
                i00 = yi * RES + xi
                row = half * 3 + k
                idxv[row, pl.ds(0 * CB + g * 16, 16)] = i00
                idxv[row, pl.ds(1 * CB + g * 16, 16)] = i00 + xs
                idxv[row, pl.ds(2 * CB + g * 16, 16)] = i00 + ys
                idxv[row, pl.ds(3 * CB + g * 16, 16)] = i00 + xs + ys
                wrow = half * 12
                gs = pl.ds(g * 16, 16)
                wv[wrow + 4 * k + 0, gs] = (1.0 - wx) * (1.0 - wy)
                wv[wrow + 4 * k + 1, gs] = wx1 * (1.0 - wy)
                wv[wrow + 4 * k + 2, gs] = (1.0 - wx) * wy1
                wv[wrow + 4 * k + 3, gs] = wx1 * wy1

    def fire(half):
        for k in range(3):
            for t in range(4):
                pltpu.async_copy(
                    tables[k].at[idxv.at[half * 3 + k, pl.ds(t * CB, CB)]],
                    taps.at[pl.ds(((half * 3 + k) * 4 + t) * CB, CB)],
                    sems[half])

    def drain(half):
        for k in range(3):
            for t in range(4):
                pltpu.make_async_copy(
                    tables[k].at[idxv.at[half * 3 + k, pl.ds(t * CB, CB)]],
                    taps.at[pl.ds(((half * 3 + k) * 4 + t) * CB, CB)],
                    sems[half]).wait()

    def combine(ci, half):
        ov = outvs[half]
        for g in range(CB // 16):
            rvec = lax.iota(jnp.int32, 16) + g * 16
            rows = [rvec + (half * 12 + j) * CB for j in range(12)]
            ws = [wv[half * 12 + j, pl.ds(g * 16, 16)] for j in range(12)]

            @plsc.parallel_loop(0, C, unroll=1)
            def cbody(c, rows=rows, ws=ws):
                cvec = (jnp.full((16,), c, jnp.int32)
                        + lax.iota(jnp.int32, 16)) & (C - 1)
                f = None
                for k in range(3):
                    acc = None
                    for t in range(4):
                        j = 4 * k + t
                        v = plsc.load_gather(taps, [rows[j], cvec])
                        term = ws[j] * v
                        acc = term if acc is None else acc + term
                    f = acc if f is None else f * acc
                plsc.store_scatter(ov, [rvec, cvec], f)

        pltpu.async_copy(ov, outh.at[pl.ds(base + ci * CB, CB)],
                         osems[half])

    def out_wait(half):
        pltpu.make_async_copy(outvs[half], outh.at[pl.ds(base, CB)],
                              osems[half]).wait()

    compute_idx(0, 0)
    fire(0)

    def step_body(s, carry):
        c0 = 2 * s
        compute_idx(c0 + 1, 1)
        fire(1)
        drain(0)

        @pl.when(s > 0)
        def _():
            out_wait(0)

        combine(c0, 0)

        @pl.when(s < NCHUNK // 2 - 1)
        def _():
            compute_idx(c0 + 2, 0)
            fire(0)

        drain(1)

        @pl.when(s > 0)
        def _():
            out_wait(1)

        combine(c0 + 1, 1)
        return carry

    lax.fori_loop(0, NCHUNK // 2, step_body, 0)
    out_wait(0)
    out_wait(1)


_tri = pl.kernel(
    _tri_body,
    out_type=jax.ShapeDtypeStruct((N_PAD, C), jnp.float32),
    mesh=plsc.VectorSubcoreMesh(core_axis_name="c", subcore_axis_name="s"),
    compiler_params=pltpu.CompilerParams(needs_layout_passes=False,
                                         disable_bounds_checks=True),
    scratch_types=[
        pltpu.VMEM((PW,), jnp.float32),
        pltpu.VMEM((PW,), jnp.float32),
        pltpu.VMEM((PW,), jnp.float32),
        pltpu.VMEM((6, 4 * CB), jnp.int32),
        pltpu.VMEM((24, CB), jnp.float32),
        pltpu.VMEM((24 * CB, C), jnp.float32),
        pltpu.VMEM((CB, C), jnp.float32),
        pltpu.VMEM((CB, C), jnp.float32),
        pltpu.SemaphoreType.DMA,
        pltpu.SemaphoreType.DMA,
        pltpu.SemaphoreType.DMA,
        pltpu.SemaphoreType.DMA,
    ],
)


def kernel(mu, P0, P1, P2):
    n = mu.shape[0]
    tb0 = jnp.transpose(P0.reshape(C, RES * RES))
    tb1 = jnp.transpose(P1.reshape(C, RES * RES))
    tb2 = jnp.transpose(P2.reshape(C, RES * RES))
    mt = jnp.pad(mu, ((0, N_PAD - n), (0, 0))).T
    out = _tri(tb0, tb1, tb2, mt[0], mt[1], mt[2])
    return out[:n]


# speedup vs baseline: 2.6377x; 1.0129x over previous
import functools

import jax
import jax.numpy as jnp
from jax import lax
from jax.experimental import pallas as pl
from jax.experimental.pallas import tpu as pltpu
from jax.experimental.pallas import tpu_sc as plsc

C = 128
RES = 256
NC = 2
NS = 16
NW = NC * NS
CB = 32
NCHUNK = 100
PW = CB * NCHUNK
N_PAD = NW * PW


def _tri_body(t0, t1, t2, m0h, m1h, m2h, outh,
              m0v, m1v, m2v, idxv, wv, taps, outv0, outv1,
              sem0, sem1, osem0, osem1):
    wid = lax.axis_index("c") * NS + lax.axis_index("s")
    base = wid * PW

    pltpu.sync_copy(m0h.at[pl.ds(base, PW)], m0v)
    pltpu.sync_copy(m1h.at[pl.ds(base, PW)], m1v)
    pltpu.sync_copy(m2h.at[pl.ds(base, PW)], m2v)

    tables = (t0, t1, t2)
    sems = (sem0, sem1)
    outvs = (outv0, outv1)
    osems = (osem0, osem1)

    def compute_idx(ci, half):
        for g in range(CB // 16):
            s = ci * CB + g * 16
            mm0 = jnp.clip(m0v[pl.ds(s, 16)], 0.0, 1.0 - 1e-6) * RES
            mm1 = jnp.clip(m1v[pl.ds(s, 16)], 0.0, 1.0 - 1e-6) * RES
            mm2 = jnp.clip(m2v[pl.ds(s, 16)], 0.0, 1.0 - 1e-6) * RES
            proj = ((mm0, mm1), (mm1, mm2), (mm2, mm0))
            for k in range(3):
                px, py = proj[k]
                xi = px.astype(jnp.int32)
                yi = py.astype(jnp.int32)
                wx = px - xi.astype(jnp.float32)
                wy = py - yi.astype(jnp.float32)
                xok = xi < RES - 1
                yok = yi < RES - 1
                xs = jnp.where(xok, 1, 0)
                ys = jnp.where(yok, RES, 0)
                wx1 = jnp.where(xok, wx, 0.0)
                wy1 = jnp.where(yok, wy, 0.0)
                i00 = yi * RES + xi
                row = half * 3 + k
                idxv[row, pl.ds(0 * CB + g * 16, 16)] = i00
                idxv[row, pl.ds(1 * CB + g * 16, 16)] = i00 + xs
                idxv[row, pl.ds(2 * CB + g * 16, 16)] = i00 + ys
                idxv[row, pl.ds(3 * CB + g * 16, 16)] = i00 + xs + ys
                wrow = half * 12
                gs = pl.ds(g * 16, 16)
                wv[wrow + 4 * k + 0, gs] = (1.0 - wx) * (1.0 - wy)
                wv[wrow + 4 * k + 1, gs] = wx1 * (1.0 - wy)
                wv[wrow + 4 * k + 2, gs] = (1.0 - wx) * wy1
                wv[wrow + 4 * k + 3, gs] = wx1 * wy1

    def fire(half):
        for k in range(3):
            for t in range(4):
                pltpu.async_copy(
                    tables[k].at[idxv.at[half * 3 + k, pl.ds(t * CB, CB)]],
                    taps.at[pl.ds(((half * 3 + k) * 4 + t) * CB, CB)],
                    sems[half])

    def drain(half):
        for k in range(3):
            for t in range(4):
                pltpu.make_async_copy(
                    tables[k].at[idxv.at[half * 3 + k, pl.ds(t * CB, CB)]],
                    taps.at[pl.ds(((half * 3 + k) * 4 + t) * CB, CB)],
                    sems[half]).wait()

    def combine(ci, half):
        ov = outvs[half]
        for g in range(CB // 16):
            rvec = lax.iota(jnp.int32, 16) + g * 16
            rows = [rvec + (half * 12 + j) * CB for j in range(12)]
            ws = [wv[half * 12 + j, pl.ds(g * 16, 16)] for j in range(12)]

            @plsc.parallel_loop(0, C, unroll=2)
            def cbody(c, rows=rows, ws=ws):
                cvec = (jnp.full((16,), c, jnp.int32)
                        + lax.iota(jnp.int32, 16)) & (C - 1)
                f = None
                for k in range(3):
                    acc = None
                    for t in range(4):
                        j = 4 * k + t
                        v = plsc.load_gather(taps, [rows[j], cvec])
                        term = ws[j] * v
                        acc = term if acc is None else acc + term
                    f = acc if f is None else f * acc
                plsc.store_scatter(ov, [rvec, cvec], f)

        pltpu.async_copy(ov, outh.at[pl.ds(base + ci * CB, CB)],
                         osems[half])

    def out_wait(half):
        pltpu.make_async_copy(outvs[half], outh.at[pl.ds(base, CB)],
                              osems[half]).wait()

    compute_idx(0, 0)
    fire(0)

    def step_body(s, carry):
        c0 = 2 * s
        compute_idx(c0 + 1, 1)
        fire(1)
        drain(0)

        @pl.when(s > 0)
        def _():
            out_wait(0)

        combine(c0, 0)

        @pl.when(s < NCHUNK // 2 - 1)
        def _():
            compute_idx(c0 + 2, 0)
            fire(0)

        drain(1)

        @pl.when(s > 0)
        def _():
            out_wait(1)

        combine(c0 + 1, 1)
        return carry

    lax.fori_loop(0, NCHUNK // 2, step_body, 0)
    out_wait(0)
    out_wait(1)


_tri = pl.kernel(
    _tri_body,
    out_type=jax.ShapeDtypeStruct((N_PAD, C), jnp.float32),
    mesh=plsc.VectorSubcoreMesh(core_axis_name="c", subcore_axis_name="s"),
    compiler_params=pltpu.CompilerParams(needs_layout_passes=False,
                                         disable_bounds_checks=True),
    scratch_types=[
        pltpu.VMEM((PW,), jnp.float32),
        pltpu.VMEM((PW,), jnp.float32),
        pltpu.VMEM((PW,), jnp.float32),
        pltpu.VMEM((6, 4 * CB), jnp.int32),
        pltpu.VMEM((24, CB), jnp.float32),
        pltpu.VMEM((24 * CB, C), jnp.float32),
        pltpu.VMEM((CB, C), jnp.float32),
        pltpu.VMEM((CB, C), jnp.float32),
        pltpu.SemaphoreType.DMA,
        pltpu.SemaphoreType.DMA,
        pltpu.SemaphoreType.DMA,
        pltpu.SemaphoreType.DMA,
    ],
)


def kernel(mu, P0, P1, P2):
    n = mu.shape[0]
    tb0 = jnp.transpose(P0.reshape(C, RES * RES))
    tb1 = jnp.transpose(P1.reshape(C, RES * RES))
    tb2 = jnp.transpose(P2.reshape(C, RES * RES))
    mt = jnp.pad(mu, ((0, N_PAD - n), (0, 0))).T
    out = _tri(tb0, tb1, tb2, mt[0], mt[1], mt[2])
    return out[:n]
